# baseline (speedup 1.0000x reference)
import functools

import jax
import jax.numpy as jnp
from jax import lax
from jax.experimental import pallas as pl
from jax.experimental.pallas import tpu as pltpu
from jax.experimental.pallas import tpu_sc as plsc

VOCAB = 1000000
D = 64
H = 256
C = 128
B = 4096
S = 200

NC = 2
NS = 16
NW = NC * NS
B_PER_W = B // NW
RING = 4
LANES = 16
NV = D // LANES


def _sc_embed_sum_body(idx_hbm, table_hbm, out_hbm, idx_v, bufs, out_v, *sems):
    wid = lax.axis_index("s") * NC + lax.axis_index("c")
    base = wid * B_PER_W

    pltpu.sync_copy(idx_hbm.at[:, pl.ds(base, B_PER_W)], idx_v)

    zero = jnp.zeros((LANES,), jnp.float32)

    @plsc.parallel_loop(0, B_PER_W)
    def _(i):
        for v in range(NV):
            out_v[i, pl.ds(v * LANES, LANES)] = zero

    def fire(s, r):
        pltpu.async_copy(table_hbm.at[idx_v.at[s]], bufs.at[r], sems[r])

    for r in range(RING):
        fire(r, r)

    def group_body(g, carry):
        for r0 in range(0, RING, 2):
            for r in (r0, r0 + 1):
                s = g * RING + r
                pltpu.make_async_copy(
                    table_hbm.at[idx_v.at[s]], bufs.at[r], sems[r]
                ).wait()

            @plsc.parallel_loop(0, B_PER_W, unroll=8)
            def _(i):
                for v in range(NV):
                    sl = pl.ds(v * LANES, LANES)
                    x = bufs[r0, i, sl] + bufs[r0 + 1, i, sl]
                    plsc.addupdate(out_v.at[i, sl], x)

            for r in (r0, r0 + 1):
                s = g * RING + r

                @pl.when(s + RING < S)
                def _():
                    fire(s + RING, r)
        return carry

    lax.fori_loop(0, S // RING, group_body, 0)

    pltpu.sync_copy(out_v, out_hbm.at[pl.ds(base, B_PER_W)])


def _sc_embed_sum(idx_t, emb_table):
    mesh = plsc.VectorSubcoreMesh(core_axis_name="c", subcore_axis_name="s")
    scratch = [
        pltpu.VMEM((S, B_PER_W), jnp.int32),
        pltpu.VMEM((RING, B_PER_W, 2 * D), jnp.float32),
        pltpu.VMEM((B_PER_W, D), jnp.float32),
    ] + [pltpu.SemaphoreType.DMA] * RING
    return pl.kernel(
        _sc_embed_sum_body,
        out_type=jax.ShapeDtypeStruct((B, D), jnp.float32),
        mesh=mesh,
        scratch_types=scratch,
        compiler_params=pltpu.CompilerParams(use_tc_tiling_on_sc=True),
    )(idx_t, emb_table)


def _mlp_body(sums_ref, len_ref, w1_ref, b1_ref, w2_ref, b2_ref, out_ref):
    avg = sums_ref[...] / len_ref[...]
    h = jnp.dot(avg, w1_ref[...], preferred_element_type=jnp.float32)
    h = jnp.maximum(h + b1_ref[...], 0.0)
    out = jnp.dot(h, w2_ref[...], preferred_element_type=jnp.float32)
    out_ref[...] = out + b2_ref[...]


def _tc_mlp(sums, text_length, W1, b1, W2, b2):
    BLK = 512
    grid = (B // BLK,)
    return pl.pallas_call(
        _mlp_body,
        grid=grid,
        in_specs=[
            pl.BlockSpec((BLK, D), lambda i: (i, 0)),
            pl.BlockSpec((BLK, 1), lambda i: (i, 0)),
            pl.BlockSpec((D, H), lambda i: (0, 0)),
            pl.BlockSpec((1, H), lambda i: (0, 0)),
            pl.BlockSpec((H, C), lambda i: (0, 0)),
            pl.BlockSpec((1, C), lambda i: (0, 0)),
        ],
        out_specs=pl.BlockSpec((BLK, C), lambda i: (i, 0)),
        out_shape=jax.ShapeDtypeStruct((B, C), jnp.float32),
    )(sums, text_length.reshape(B, 1), W1, b1.reshape(1, H), W2, b2.reshape(1, C))


@jax.jit
def kernel(input_text, text_length, emb_table, W1, b1, W2, b2):
    idx_t = input_text.astype(jnp.int32).T
    table_pad = jnp.pad(emb_table, ((0, 0), (0, D)))
    sums = _sc_embed_sum(idx_t, table_pad)
    return _tc_mlp(sums, text_length, W1, b1, W2, b2)


# speedup vs baseline: 1.0658x; 1.0658x over previous
import functools

import jax
import jax.numpy as jnp
from jax import lax
from jax.experimental import pallas as pl
from jax.experimental.pallas import tpu as pltpu
from jax.experimental.pallas import tpu_sc as plsc

VOCAB = 1000000
D = 64
H = 256
C = 128
B = 4096
S = 200

NC = 2
NS = 16
NW = NC * NS
B_PER_W = B // NW
RING = 4
LANES = 16
NV = D // LANES


def _sc_embed_sum_body(idx_hbm, table_hbm, out_hbm, idx_v, bufs, out_v, *sems):
    wid = lax.axis_index("s") * NC + lax.axis_index("c")
    base = wid * B_PER_W

    pltpu.sync_copy(idx_hbm.at[:, pl.ds(base, B_PER_W)], idx_v)

    zero = jnp.zeros((LANES,), jnp.float32)

    @plsc.parallel_loop(0, B_PER_W)
    def _(i):
        for v in range(NV):
            out_v[i, pl.ds(v * LANES, LANES)] = zero

    def fire(s, r):
        pltpu.async_copy(table_hbm.at[idx_v.at[s]], bufs.at[r], sems[r])

    for r in range(RING):
        fire(r, r)

    def group_body(g, carry):
        for r in range(RING):
            s = g * RING + r
            pltpu.make_async_copy(
                table_hbm.at[idx_v.at[s]], bufs.at[r], sems[r]
            ).wait()

            @plsc.parallel_loop(0, B_PER_W, unroll=8)
            def _(i):
                for v in range(NV):
                    x = bufs[r, i, pl.ds(v * LANES, LANES)]
                    plsc.addupdate(out_v.at[i, pl.ds(v * LANES, LANES)], x)

            @pl.when(s + RING < S)
            def _():
                fire(s + RING, r)
        return carry

    lax.fori_loop(0, S // RING, group_body, 0)

    pltpu.sync_copy(out_v, out_hbm.at[pl.ds(base, B_PER_W)])


def _sc_embed_sum(idx_t, emb_table):
    mesh = plsc.VectorSubcoreMesh(core_axis_name="c", subcore_axis_name="s")
    scratch = [
        pltpu.VMEM((S, B_PER_W), jnp.int32),
        pltpu.VMEM((RING, B_PER_W, 2 * D), jnp.float32),
        pltpu.VMEM((B_PER_W, D), jnp.float32),
    ] + [pltpu.SemaphoreType.DMA] * RING
    return pl.kernel(
        _sc_embed_sum_body,
        out_type=jax.ShapeDtypeStruct((B, D), jnp.float32),
        mesh=mesh,
        scratch_types=scratch,
        compiler_params=pltpu.CompilerParams(use_tc_tiling_on_sc=True),
    )(idx_t, emb_table)


def _mlp_body(sums_ref, len_ref, w1_ref, b1_ref, w2_ref, b2_ref, out_ref):
    avg = sums_ref[...] / len_ref[...]
    h = jnp.dot(avg, w1_ref[...], preferred_element_type=jnp.float32)
    h = jnp.maximum(h + b1_ref[...], 0.0)
    out = jnp.dot(h, w2_ref[...], preferred_element_type=jnp.float32)
    out_ref[...] = out + b2_ref[...]


def _tc_mlp(sums, text_length, W1, b1, W2, b2):
    BLK = 512
    grid = (B // BLK,)
    return pl.pallas_call(
        _mlp_body,
        grid=grid,
        in_specs=[
            pl.BlockSpec((BLK, D), lambda i: (i, 0)),
            pl.BlockSpec((BLK, 1), lambda i: (i, 0)),
            pl.BlockSpec((D, H), lambda i: (0, 0)),
            pl.BlockSpec((1, H), lambda i: (0, 0)),
            pl.BlockSpec((H, C), lambda i: (0, 0)),
            pl.BlockSpec((1, C), lambda i: (0, 0)),
        ],
        out_specs=pl.BlockSpec((BLK, C), lambda i: (i, 0)),
        out_shape=jax.ShapeDtypeStruct((B, C), jnp.float32),
    )(sums, text_length.reshape(B, 1), W1, b1.reshape(1, H), W2, b2.reshape(1, C))


@jax.jit
def kernel(input_text, text_length, emb_table, W1, b1, W2, b2):
    idx_t = input_text.astype(jnp.int32).T
    table_pad = jnp.pad(emb_table, ((0, 0), (0, D)))
    sums = _sc_embed_sum(idx_t, table_pad)
    return _tc_mlp(sums, text_length, W1, b1, W2, b2)
